# baseline (speedup 1.0000x reference)
import functools

import jax
import jax.numpy as jnp
from jax import lax
from jax.experimental import pallas as pl
from jax.experimental.pallas import tpu as pltpu
from jax.experimental.pallas import tpu_sc as plsc

_VOCAB = 1000000
_EMBED = 128
_HIDDEN = 128
_OUTPUTS = 91
_BATCH = 16384
_CTX = 6
_N_ROWS = _BATCH * _CTX

_INFO = plsc.get_sparse_core_info()
_NC = _INFO.num_cores
_NS = _INFO.num_subcores
_NW = _NC * _NS

_CH = 384

_sc_mesh = plsc.VectorSubcoreMesh(core_axis_name="c", subcore_axis_name="s")


def _make_sc_gather(n_rows):
    b_per_w = n_rows // _NW
    n_ch = b_per_w // _CH

    @functools.partial(
        pl.kernel,
        mesh=_sc_mesh,
        out_type=jax.ShapeDtypeStruct((n_rows, _EMBED), jnp.float32),
        scratch_types=[
            pltpu.VMEM((b_per_w,), jnp.int32),
            pltpu.VMEM((2, _CH, _EMBED), jnp.float32),
            pltpu.SemaphoreType.DMA,
            pltpu.SemaphoreType.DMA,
        ],
    )
    def _sc_gather(idx_hbm, table_hbm, out_hbm, idx_v, rows_v, sem0, sem1):
        wid = lax.axis_index("s") * _NC + lax.axis_index("c")
        base = wid * b_per_w
        pltpu.sync_copy(idx_hbm.at[pl.ds(base, b_per_w)], idx_v)
        sems = (sem0, sem1)
        pending = pltpu.async_copy(
            table_hbm.at[idx_v.at[pl.ds(0, _CH)]], rows_v.at[0], sems[0]
        )
        for c in range(n_ch):
            nxt = None
            if c + 1 < n_ch:
                nxt = pltpu.async_copy(
                    table_hbm.at[idx_v.at[pl.ds((c + 1) * _CH, _CH)]],
                    rows_v.at[(c + 1) % 2],
                    sems[(c + 1) % 2],
                )
            pending.wait()
            pltpu.sync_copy(
                rows_v.at[c % 2], out_hbm.at[pl.ds(base + c * _CH, _CH)]
            )
            pending = nxt

    return _sc_gather


def _mlp_body(x_ref, w1_ref, b1_ref, w2_ref, b2_ref, o_ref):
    h = jnp.dot(x_ref[0], w1_ref[0], preferred_element_type=jnp.float32)
    for j in range(1, _CTX):
        h = h + jnp.dot(x_ref[j], w1_ref[j], preferred_element_type=jnp.float32)
    h = jnp.maximum(h + b1_ref[...], 0.0)
    o_ref[...] = (
        jnp.dot(h, w2_ref[...], preferred_element_type=jnp.float32) + b2_ref[...]
    )


def _mlp_body_chain(x_ref, w1_ref, b1_ref, w2_ref, b2_ref, prev_ref, o_ref):
    del prev_ref
    _mlp_body(x_ref, w1_ref, b1_ref, w2_ref, b2_ref, o_ref)


_BM = 1024


def _mlp_split(x, W1, b1, W2, b2, prev, base_row, h):
    nblk = h // _BM
    base = base_row // _BM
    common = dict(
        grid=(nblk,),
        out_specs=pl.BlockSpec((_BM, _OUTPUTS), lambda i: (i + base, 0)),
        out_shape=jax.ShapeDtypeStruct((_BATCH, _OUTPUTS), jnp.float32),
    )
    in_specs = [
        pl.BlockSpec((_CTX, _BM, _EMBED), lambda i: (0, i, 0)),
        pl.BlockSpec((_CTX, _EMBED, _HIDDEN), lambda i: (0, 0, 0)),
        pl.BlockSpec((1, _HIDDEN), lambda i: (0, 0)),
        pl.BlockSpec((_HIDDEN, _OUTPUTS), lambda i: (0, 0)),
        pl.BlockSpec((1, _OUTPUTS), lambda i: (0, 0)),
    ]
    args = (x, W1, b1.reshape(1, -1), W2, b2.reshape(1, -1))
    if prev is None:
        return pl.pallas_call(_mlp_body, in_specs=in_specs, **common)(*args)
    return pl.pallas_call(
        _mlp_body_chain,
        in_specs=in_specs + [pl.BlockSpec(memory_space=pl.ANY)],
        input_output_aliases={5: 0},
        **common,
    )(*args, prev)


_SPLITS = (8192, 6144, 2048)
_sc_gathers = {h: _make_sc_gather(_CTX * h) for h in set(_SPLITS)}


def kernel(inputs, table, W1, b1, W2, b2):
    w1 = W1.reshape(_CTX, _EMBED, _HIDDEN)
    xs = []
    base = 0
    for h in _SPLITS:
        idx_t = inputs[base:base + h].T.reshape(-1)
        xs.append(_sc_gathers[h](idx_t, table).reshape(_CTX, h, _EMBED))
        base += h
    out = jnp.zeros((_BATCH, _OUTPUTS), jnp.float32)
    base = 0
    for x, h in zip(xs, _SPLITS):
        out = _mlp_split(x, w1, b1, W2, b2, out, base, h)
        base += h
    return out


# speedup vs baseline: 1.0191x; 1.0191x over previous
import functools

import jax
import jax.numpy as jnp
from jax import lax
from jax.experimental import pallas as pl
from jax.experimental.pallas import tpu as pltpu
from jax.experimental.pallas import tpu_sc as plsc

_VOCAB = 1000000
_EMBED = 128
_HIDDEN = 128
_OUTPUTS = 91
_BATCH = 16384
_CTX = 6
_N_ROWS = _BATCH * _CTX

_INFO = plsc.get_sparse_core_info()
_NC = _INFO.num_cores
_NS = _INFO.num_subcores
_NW = _NC * _NS

_CH = 192

_sc_mesh = plsc.VectorSubcoreMesh(core_axis_name="c", subcore_axis_name="s")


def _make_sc_gather(n_rows):
    b_per_w = n_rows // _NW
    n_ch = b_per_w // _CH

    @functools.partial(
        pl.kernel,
        mesh=_sc_mesh,
        out_type=jax.ShapeDtypeStruct((n_rows, _EMBED), jnp.float32),
        scratch_types=[
            pltpu.VMEM((b_per_w,), jnp.int32),
            pltpu.VMEM((3, _CH, _EMBED), jnp.float32),
            pltpu.SemaphoreType.DMA,
            pltpu.SemaphoreType.DMA,
            pltpu.SemaphoreType.DMA,
            pltpu.SemaphoreType.DMA,
            pltpu.SemaphoreType.DMA,
            pltpu.SemaphoreType.DMA,
        ],
    )
    def _sc_gather(idx_hbm, table_hbm, out_hbm, idx_v, rows_v,
                   g0, g1, g2, s0, s1, s2):
        wid = lax.axis_index("s") * _NC + lax.axis_index("c")
        base = wid * b_per_w
        pltpu.sync_copy(idx_hbm.at[pl.ds(base, b_per_w)], idx_v)
        gsem = (g0, g1, g2)
        ssem = (s0, s1, s2)

        def start_gather(c):
            return pltpu.async_copy(
                table_hbm.at[idx_v.at[pl.ds(c * _CH, _CH)]],
                rows_v.at[c % 3],
                gsem[c % 3],
            )

        pend_g = {c: start_gather(c) for c in range(min(2, n_ch))}
        pend_s = {}
        for c in range(n_ch):
            pend_g.pop(c).wait()
            pend_s[c] = pltpu.async_copy(
                rows_v.at[c % 3],
                out_hbm.at[pl.ds(base + c * _CH, _CH)],
                ssem[c % 3],
            )
            nxt = c + 2
            if nxt < n_ch:
                if nxt - 3 >= 0:
                    pend_s.pop(nxt - 3).wait()
                pend_g[nxt] = start_gather(nxt)
        for c in sorted(pend_s):
            pend_s.pop(c).wait()

    return _sc_gather


def _mlp_body(x_ref, w1_ref, b1_ref, w2_ref, b2_ref, o_ref):
    h = jnp.dot(x_ref[0], w1_ref[0], preferred_element_type=jnp.float32)
    for j in range(1, _CTX):
        h = h + jnp.dot(x_ref[j], w1_ref[j], preferred_element_type=jnp.float32)
    h = jnp.maximum(h + b1_ref[...], 0.0)
    o_ref[...] = (
        jnp.dot(h, w2_ref[...], preferred_element_type=jnp.float32) + b2_ref[...]
    )


def _mlp_body_chain(x_ref, w1_ref, b1_ref, w2_ref, b2_ref, prev_ref, o_ref):
    del prev_ref
    _mlp_body(x_ref, w1_ref, b1_ref, w2_ref, b2_ref, o_ref)


_BM = 1024


def _mlp_split(x, W1, b1, W2, b2, prev, base_row, h):
    nblk = h // _BM
    base = base_row // _BM
    common = dict(
        grid=(nblk,),
        out_specs=pl.BlockSpec((_BM, _OUTPUTS), lambda i: (i + base, 0)),
        out_shape=jax.ShapeDtypeStruct((_BATCH, _OUTPUTS), jnp.float32),
    )
    in_specs = [
        pl.BlockSpec((_CTX, _BM, _EMBED), lambda i: (0, i, 0)),
        pl.BlockSpec((_CTX, _EMBED, _HIDDEN), lambda i: (0, 0, 0)),
        pl.BlockSpec((1, _HIDDEN), lambda i: (0, 0)),
        pl.BlockSpec((_HIDDEN, _OUTPUTS), lambda i: (0, 0)),
        pl.BlockSpec((1, _OUTPUTS), lambda i: (0, 0)),
    ]
    args = (x, W1, b1.reshape(1, -1), W2, b2.reshape(1, -1))
    if prev is None:
        return pl.pallas_call(_mlp_body, in_specs=in_specs, **common)(*args)
    return pl.pallas_call(
        _mlp_body_chain,
        in_specs=in_specs + [pl.BlockSpec(memory_space=pl.ANY)],
        input_output_aliases={5: 0},
        **common,
    )(*args, prev)


_SPLITS = (8192, 6144, 2048)
_sc_gathers = {h: _make_sc_gather(_CTX * h) for h in set(_SPLITS)}


def kernel(inputs, table, W1, b1, W2, b2):
    w1 = W1.reshape(_CTX, _EMBED, _HIDDEN)
    xs = []
    base = 0
    for h in _SPLITS:
        idx_t = inputs[base:base + h].T.reshape(-1)
        xs.append(_sc_gathers[h](idx_t, table).reshape(_CTX, h, _EMBED))
        base += h
    out = None
    base = 0
    for x, h in zip(xs, _SPLITS):
        out = _mlp_split(x, w1, b1, W2, b2, out, base, h)
        base += h
    return out


# speedup vs baseline: 1.0344x; 1.0151x over previous
import functools

import jax
import jax.numpy as jnp
from jax import lax
from jax.experimental import pallas as pl
from jax.experimental.pallas import tpu as pltpu
from jax.experimental.pallas import tpu_sc as plsc

_VOCAB = 1000000
_EMBED = 128
_HIDDEN = 128
_OUTPUTS = 91
_BATCH = 16384
_CTX = 6
_N_ROWS = _BATCH * _CTX

_INFO = plsc.get_sparse_core_info()
_NC = _INFO.num_cores
_NS = _INFO.num_subcores
_NW = _NC * _NS

_CH = 384

_sc_mesh = plsc.VectorSubcoreMesh(core_axis_name="c", subcore_axis_name="s")


def _make_sc_gather(n_rows):
    b_per_w = n_rows // _NW
    n_ch = b_per_w // _CH

    @functools.partial(
        pl.kernel,
        mesh=_sc_mesh,
        out_type=jax.ShapeDtypeStruct((n_rows, _EMBED), jnp.float32),
        scratch_types=[
            pltpu.VMEM((b_per_w,), jnp.int32),
            pltpu.VMEM((2, _CH, _EMBED), jnp.float32),
            pltpu.SemaphoreType.DMA,
            pltpu.SemaphoreType.DMA,
        ],
    )
    def _sc_gather(idx_hbm, table_hbm, out_hbm, idx_v, rows_v, sem0, sem1):
        wid = lax.axis_index("s") * _NC + lax.axis_index("c")
        base = wid * b_per_w
        pltpu.sync_copy(idx_hbm.at[pl.ds(base, b_per_w)], idx_v)
        sems = (sem0, sem1)
        pending = pltpu.async_copy(
            table_hbm.at[idx_v.at[pl.ds(0, _CH)]], rows_v.at[0], sems[0]
        )
        for c in range(n_ch):
            nxt = None
            if c + 1 < n_ch:
                nxt = pltpu.async_copy(
                    table_hbm.at[idx_v.at[pl.ds((c + 1) * _CH, _CH)]],
                    rows_v.at[(c + 1) % 2],
                    sems[(c + 1) % 2],
                )
            pending.wait()
            pltpu.sync_copy(
                rows_v.at[c % 2], out_hbm.at[pl.ds(base + c * _CH, _CH)]
            )
            pending = nxt

    return _sc_gather


def _mlp_body(x_ref, w1_ref, b1_ref, w2_ref, b2_ref, o_ref):
    h = jnp.dot(x_ref[0], w1_ref[0], preferred_element_type=jnp.float32)
    for j in range(1, _CTX):
        h = h + jnp.dot(x_ref[j], w1_ref[j], preferred_element_type=jnp.float32)
    h = jnp.maximum(h + b1_ref[...], 0.0)
    o_ref[...] = (
        jnp.dot(h, w2_ref[...], preferred_element_type=jnp.float32) + b2_ref[...]
    )


def _mlp_body_chain(x_ref, w1_ref, b1_ref, w2_ref, b2_ref, prev_ref, o_ref):
    del prev_ref
    _mlp_body(x_ref, w1_ref, b1_ref, w2_ref, b2_ref, o_ref)


_BM = 1024


def _mlp_split(x, W1, b1, W2, b2, prev, base_row, h):
    nblk = h // _BM
    base = base_row // _BM
    common = dict(
        grid=(nblk,),
        out_specs=pl.BlockSpec((_BM, _OUTPUTS), lambda i: (i + base, 0)),
        out_shape=jax.ShapeDtypeStruct((_BATCH, _OUTPUTS), jnp.float32),
    )
    in_specs = [
        pl.BlockSpec((_CTX, _BM, _EMBED), lambda i: (0, i, 0)),
        pl.BlockSpec((_CTX, _EMBED, _HIDDEN), lambda i: (0, 0, 0)),
        pl.BlockSpec((1, _HIDDEN), lambda i: (0, 0)),
        pl.BlockSpec((_HIDDEN, _OUTPUTS), lambda i: (0, 0)),
        pl.BlockSpec((1, _OUTPUTS), lambda i: (0, 0)),
    ]
    args = (x, W1, b1.reshape(1, -1), W2, b2.reshape(1, -1))
    if prev is None:
        return pl.pallas_call(_mlp_body, in_specs=in_specs, **common)(*args)
    return pl.pallas_call(
        _mlp_body_chain,
        in_specs=in_specs + [pl.BlockSpec(memory_space=pl.ANY)],
        input_output_aliases={5: 0},
        **common,
    )(*args, prev)


_SPLITS = (8192, 6144, 2048)
_sc_gathers = {h: _make_sc_gather(_CTX * h) for h in set(_SPLITS)}


def kernel(inputs, table, W1, b1, W2, b2):
    w1 = W1.reshape(_CTX, _EMBED, _HIDDEN)
    xs = []
    base = 0
    for h in _SPLITS:
        idx_t = inputs[base:base + h].T.reshape(-1)
        xs.append(_sc_gathers[h](idx_t, table).reshape(_CTX, h, _EMBED))
        base += h
    out = None
    base = 0
    for x, h in zip(xs, _SPLITS):
        out = _mlp_split(x, w1, b1, W2, b2, out, base, h)
        base += h
    return out
